# baseline (speedup 1.0000x reference)
import functools

import jax
import jax.numpy as jnp
from jax import lax
from jax.experimental import pallas as pl
from jax.experimental.pallas import tpu as pltpu
from jax.experimental.pallas import tpu_sc as plsc

_NUM_MODELS = 1000000
_DIM = 64
_TEXT_DIM = 128
_BATCH = 16384

_INFO = plsc.get_sparse_core_info()
_NC, _NS = _INFO.num_cores, _INFO.num_subcores
_NW = _NC * _NS
_B_PER_W = _BATCH // _NW


def _sc_gather_kernel(table_hbm, idx_hbm, out_hbm, idx_v, rows_v, sem):
    wid = lax.axis_index("s") * _NC + lax.axis_index("c")
    base = wid * _B_PER_W
    pltpu.sync_copy(idx_hbm.at[pl.ds(base, _B_PER_W)], idx_v)
    pltpu.async_copy(table_hbm.at[idx_v], rows_v, sem).wait()
    pltpu.sync_copy(rows_v, out_hbm.at[pl.ds(base, _B_PER_W)])


@jax.jit
def _sc_gather(table, idx):
    mesh = plsc.VectorSubcoreMesh(core_axis_name="c", subcore_axis_name="s")
    k = functools.partial(
        pl.kernel,
        mesh=mesh,
        out_type=jax.ShapeDtypeStruct((_BATCH, _DIM), jnp.float32),
        scratch_types=[
            pltpu.VMEM((_B_PER_W,), jnp.int32),
            pltpu.VMEM((_B_PER_W, _DIM), jnp.float32),
            pltpu.SemaphoreType.DMA,
        ],
        compiler_params=pltpu.CompilerParams(use_tc_tiling_on_sc=False),
    )(_sc_gather_kernel)
    return k(table, idx)


_TC_BLOCK = 2048


def _tc_dense_kernel(prompt_ref, w_text_ref, w_cls_ref, rows_ref, out_ref):
    t = lax.dot_general(
        prompt_ref[...], w_text_ref[...],
        dimension_numbers=(((1,), (1,)), ((), ())),
        preferred_element_type=jnp.float32,
    )
    prod = rows_ref[...] * t * w_cls_ref[...]
    pred = jnp.sum(prod, axis=1, keepdims=True)
    out_ref[...] = jax.nn.sigmoid(pred)


@jax.jit
def _tc_dense(prompt, w_text, w_cls, rows):
    grid = _BATCH // _TC_BLOCK
    out = pl.pallas_call(
        _tc_dense_kernel,
        grid=(grid,),
        in_specs=[
            pl.BlockSpec((_TC_BLOCK, _TEXT_DIM), lambda i: (i, 0)),
            pl.BlockSpec((_DIM, _TEXT_DIM), lambda i: (0, 0)),
            pl.BlockSpec((1, _DIM), lambda i: (0, 0)),
            pl.BlockSpec((_TC_BLOCK, _DIM), lambda i: (i, 0)),
        ],
        out_specs=pl.BlockSpec((_TC_BLOCK, 1), lambda i: (i, 0)),
        out_shape=jax.ShapeDtypeStruct((_BATCH, 1), jnp.float32),
    )(prompt, w_text, w_cls, rows)
    return out.reshape(_BATCH)


def kernel(model_id, prompt_embedding, model_embed_table, W_text, W_cls):
    idx = model_id.astype(jnp.int32)
    rows = _sc_gather(model_embed_table, idx)
    return _tc_dense(prompt_embedding, W_text, W_cls, rows)
